# Initial kernel scaffold; baseline (speedup 1.0000x reference)
import jax
import jax.numpy as jnp
from jax.experimental import pallas as pl


def kernel(x, edge_index, batch, W0, b0, W1, b1, W2, b2, Wh1, bh1, Wh2, bh2):
    raise NotImplementedError("write your pallas kernel here")


# speedup vs baseline: 6.5934x; 6.5934x over previous
import functools

import jax
import jax.numpy as jnp
from jax import lax
from jax.experimental import pallas as pl
from jax.experimental.pallas import tpu as pltpu
from jax.experimental.pallas import tpu_sc as plsc

NC = 2
NS = 16
NW = NC * NS
CHUNK = 128
ZR = 80
PADROWS = 80


def _sc_mesh():
    return plsc.VectorSubcoreMesh(core_axis_name="c", subcore_axis_name="s")


def _make_gather(N, EP, D):
    NCH = EP // CHUNK
    KMAX = NCH // NW

    @functools.partial(
        pl.kernel,
        out_type=jax.ShapeDtypeStruct((EP, D), jnp.float32),
        mesh=_sc_mesh(),
        scratch_types=[
            pltpu.VMEM((CHUNK,), jnp.int32),
            pltpu.VMEM((CHUNK, D), jnp.float32),
            pltpu.SemaphoreType.DMA,
        ],
    )
    def gath(s_hbm, row_hbm, msg, rowbuf, gbuf, sem):
        c = lax.axis_index("c")
        sid = lax.axis_index("s")
        w = c * NS + sid

        @pl.loop(0, KMAX)
        def ebody(k):
            eb = (w + NW * k) * CHUNK
            pltpu.sync_copy(row_hbm.at[pl.ds(eb, CHUNK)], rowbuf)
            pltpu.async_copy(s_hbm.at[rowbuf], gbuf, sem).wait()
            pltpu.sync_copy(gbuf, msg.at[pl.ds(eb, CHUNK)])

    return gath


def _make_scatter(N, EP, D):
    NCH = EP // CHUNK
    KMAX = NCH // NW
    NA = N + PADROWS
    NZ = NA // ZR
    NO = N // ZR
    JMAX = (NZ + NS - 1) // NS

    @functools.partial(
        pl.kernel,
        out_type=(jax.ShapeDtypeStruct((N, D), jnp.float32),
                  jax.ShapeDtypeStruct((N, D), jnp.float32)),
        mesh=_sc_mesh(),
        scratch_types=[
            pltpu.VMEM((CHUNK,), jnp.int32),
            pltpu.VMEM((CHUNK, D), jnp.float32),
            pltpu.VMEM((ZR, D), jnp.float32),
            pltpu.VMEM_SHARED((N + PADROWS, D), jnp.float32),
        ],
    )
    def scat(msg_hbm, col_hbm, zeros_hbm, out_a, out_b,
             colbuf, gbuf, zbuf, acc):
        c = lax.axis_index("c")
        sid = lax.axis_index("s")
        w = c * NS + sid

        pltpu.sync_copy(zeros_hbm, zbuf)

        @pl.loop(0, JMAX)
        def zbody(j):
            ch = sid + NS * j

            @pl.when(ch < NZ)
            def _():
                pltpu.sync_copy(zbuf, acc.at[pl.ds(ch * ZR, ZR)])

        plsc.subcore_barrier()

        @pl.loop(0, KMAX)
        def ebody(k):
            eb = (w + NW * k) * CHUNK
            pltpu.sync_copy(col_hbm.at[pl.ds(eb, CHUNK)], colbuf)
            pltpu.sync_copy(msg_hbm.at[pl.ds(eb, CHUNK)], gbuf)
            pltpu.sync_copy(gbuf, acc.at[colbuf], add=True)

        plsc.subcore_barrier()

        @pl.loop(0, JMAX)
        def obody(j):
            ch = sid + NS * j

            @pl.when(ch < NO)
            def _():
                rows = pl.ds(ch * ZR, ZR)
                pltpu.sync_copy(acc.at[rows], zbuf)

                @pl.when(c == 0)
                def _():
                    pltpu.sync_copy(zbuf, out_a.at[rows])

                @pl.when(c == 1)
                def _():
                    pltpu.sync_copy(zbuf, out_b.at[rows])

    return scat


def _make_deg(N, EP, D):
    NCH = EP // CHUNK
    KMAX = NCH // NW
    NA = N + PADROWS
    NZ = NA // ZR
    NO = N // ZR
    JMAX = (NZ + NS - 1) // NS

    @functools.partial(
        pl.kernel,
        out_type=(jax.ShapeDtypeStruct((N, D), jnp.float32),
                  jax.ShapeDtypeStruct((N, D), jnp.float32)),
        mesh=_sc_mesh(),
        scratch_types=[
            pltpu.VMEM((CHUNK,), jnp.int32),
            pltpu.VMEM((CHUNK, D), jnp.float32),
            pltpu.VMEM((ZR, D), jnp.float32),
            pltpu.VMEM_SHARED((N + PADROWS, D), jnp.float32),
        ],
    )
    def deg(col_hbm, ones_hbm, zeros_hbm, out_a, out_b,
            colbuf, onesbuf, zbuf, acc):
        c = lax.axis_index("c")
        sid = lax.axis_index("s")
        w = c * NS + sid

        pltpu.sync_copy(ones_hbm, onesbuf)
        pltpu.sync_copy(zeros_hbm, zbuf)

        @pl.loop(0, JMAX)
        def zbody(j):
            ch = sid + NS * j

            @pl.when(ch < NZ)
            def _():
                pltpu.sync_copy(zbuf, acc.at[pl.ds(ch * ZR, ZR)])

        plsc.subcore_barrier()

        @pl.loop(0, KMAX)
        def ebody(k):
            eb = (w + NW * k) * CHUNK
            pltpu.sync_copy(col_hbm.at[pl.ds(eb, CHUNK)], colbuf)
            pltpu.sync_copy(onesbuf, acc.at[colbuf], add=True)

        plsc.subcore_barrier()

        @pl.loop(0, JMAX)
        def obody(j):
            ch = sid + NS * j

            @pl.when(ch < NO)
            def _():
                rows = pl.ds(ch * ZR, ZR)
                pltpu.sync_copy(acc.at[rows], zbuf)

                @pl.when(c == 0)
                def _():
                    pltpu.sync_copy(zbuf, out_a.at[rows])

                @pl.when(c == 1)
                def _():
                    pltpu.sync_copy(zbuf, out_b.at[rows])

    return deg


def _dinv_from(da_ref, db_ref):
    return lax.rsqrt(da_ref[:, 0:1] + db_ref[:, 0:1] + 1.0)


def _mm_first(x, W, b, dega, degb, BLK=1000):
    N, D = x.shape

    def body(x_ref, w_ref, b_ref, da_ref, db_ref, o_ref):
        dinv = _dinv_from(da_ref, db_ref)
        z = jnp.dot(x_ref[:, :], w_ref[:, :],
                    preferred_element_type=jnp.float32) + b_ref[0:1, :]
        o_ref[:, :] = z * dinv

    return pl.pallas_call(
        body,
        grid=(N // BLK,),
        in_specs=[
            pl.BlockSpec((BLK, D), lambda i: (i, 0)),
            pl.BlockSpec((D, D), lambda i: (0, 0)),
            pl.BlockSpec((1, D), lambda i: (0, 0)),
            pl.BlockSpec((BLK, D), lambda i: (i, 0)),
            pl.BlockSpec((BLK, D), lambda i: (i, 0)),
        ],
        out_specs=pl.BlockSpec((BLK, D), lambda i: (i, 0)),
        out_shape=jax.ShapeDtypeStruct((N, D), jnp.float32),
    )(x, W, b, dega, degb)


def _mm_mid(pa, pb, s_prev, W, b, dega, degb, BLK=1000):
    N, D = s_prev.shape

    def body(pa_ref, pb_ref, s_ref, w_ref, b_ref, da_ref, db_ref, o_ref):
        dinv = _dinv_from(da_ref, db_ref)
        h = jnp.maximum((pa_ref[:, :] + pb_ref[:, :] + s_ref[:, :]) * dinv, 0.0)
        z = jnp.dot(h, w_ref[:, :],
                    preferred_element_type=jnp.float32) + b_ref[0:1, :]
        o_ref[:, :] = z * dinv

    return pl.pallas_call(
        body,
        grid=(N // BLK,),
        in_specs=[
            pl.BlockSpec((BLK, D), lambda i: (i, 0)),
            pl.BlockSpec((BLK, D), lambda i: (i, 0)),
            pl.BlockSpec((BLK, D), lambda i: (i, 0)),
            pl.BlockSpec((D, D), lambda i: (0, 0)),
            pl.BlockSpec((1, D), lambda i: (0, 0)),
            pl.BlockSpec((BLK, D), lambda i: (i, 0)),
            pl.BlockSpec((BLK, D), lambda i: (i, 0)),
        ],
        out_specs=pl.BlockSpec((BLK, D), lambda i: (i, 0)),
        out_shape=jax.ShapeDtypeStruct((N, D), jnp.float32),
    )(pa, pb, s_prev, W, b, dega, degb)


def _final(pa, pb, s_prev, dega, degb, batch3d, Wh1, bh1, Wh2p, bh2p, BLK=1000):
    N, D = s_prev.shape
    GP = 128
    nblk = N // BLK

    def body(pa_ref, pb_ref, s_ref, da_ref, db_ref, bt_ref,
             wh1_ref, bh1_ref, wh2_ref, bh2_ref, o_ref, pool_acc, cnt_acc):
        i = pl.program_id(0)

        @pl.when(i == 0)
        def _():
            pool_acc[:, :] = jnp.zeros((GP, D), jnp.float32)
            cnt_acc[:, :] = jnp.zeros((GP, D), jnp.float32)

        dinv = _dinv_from(da_ref, db_ref)
        h = jnp.maximum((pa_ref[:, :] + pb_ref[:, :] + s_ref[:, :]) * dinv, 0.0)
        bt = jnp.broadcast_to(bt_ref[0], (GP, BLK))
        gid = lax.broadcasted_iota(jnp.int32, (GP, BLK), 0)
        onehot_t = jnp.where(bt == gid, 1.0, 0.0)
        pool_acc[:, :] += lax.dot_general(
            onehot_t, h, (((1,), (0,)), ((), ())),
            preferred_element_type=jnp.float32)
        cnt_acc[:, :] += lax.dot_general(
            onehot_t, jnp.ones((BLK, D), jnp.float32), (((1,), (0,)), ((), ())),
            preferred_element_type=jnp.float32)

        @pl.when(i == nblk - 1)
        def _():
            g = pool_acc[:, :] / jnp.maximum(cnt_acc[:, :], 1.0)
            g1 = jnp.maximum(
                jnp.dot(g, wh1_ref[:, :],
                        preferred_element_type=jnp.float32) + bh1_ref[0:1, :],
                0.0)
            o_ref[:, :] = jnp.dot(g1, wh2_ref[:, :],
                                  preferred_element_type=jnp.float32) + bh2_ref[0:1, :]

    return pl.pallas_call(
        body,
        grid=(nblk,),
        in_specs=[
            pl.BlockSpec((BLK, D), lambda i: (i, 0)),
            pl.BlockSpec((BLK, D), lambda i: (i, 0)),
            pl.BlockSpec((BLK, D), lambda i: (i, 0)),
            pl.BlockSpec((BLK, D), lambda i: (i, 0)),
            pl.BlockSpec((BLK, D), lambda i: (i, 0)),
            pl.BlockSpec((1, 1, BLK), lambda i: (i, 0, 0)),
            pl.BlockSpec((D, D), lambda i: (0, 0)),
            pl.BlockSpec((1, D), lambda i: (0, 0)),
            pl.BlockSpec((D, GP), lambda i: (0, 0)),
            pl.BlockSpec((1, GP), lambda i: (0, 0)),
        ],
        out_specs=pl.BlockSpec((GP, D), lambda i: (0, 0)),
        out_shape=jax.ShapeDtypeStruct((GP, D), jnp.float32),
        scratch_shapes=[
            pltpu.VMEM((GP, D), jnp.float32),
            pltpu.VMEM((GP, D), jnp.float32),
        ],
    )(pa, pb, s_prev, dega, degb, batch3d, Wh1, bh1, Wh2p, bh2p)


def kernel(x, edge_index, batch, W0, b0, W1, b1, W2, b2, Wh1, bh1, Wh2, bh2):
    N, D = x.shape
    E = edge_index.shape[1]
    G = 64
    EP = ((E + NW * CHUNK - 1) // (NW * CHUNK)) * (NW * CHUNK)
    pad = EP - E
    row = jnp.concatenate([edge_index[0], jnp.zeros((pad,), jnp.int32)])
    col = jnp.concatenate([edge_index[1], jnp.full((pad,), N, jnp.int32)])

    zeros_d = jnp.zeros((ZR, D), jnp.float32)
    ones_d = jnp.ones((CHUNK, D), jnp.float32)
    batch3d = batch.reshape(N // 1000, 1, 1000)
    b0r = b0.reshape(1, D)
    b1r = b1.reshape(1, D)
    b2r = b2.reshape(1, D)
    bh1r = bh1.reshape(1, D)
    Wh2p = jnp.pad(Wh2, ((0, 0), (0, 128 - Wh2.shape[1])))
    bh2p = jnp.broadcast_to(bh2.reshape(1, 1), (1, 128))

    deg = _make_deg(N, EP, D)
    gath = _make_gather(N, EP, D)
    scat = _make_scatter(N, EP, D)

    dega, degb = deg(col, ones_d, zeros_d)

    s0 = _mm_first(x, W0, b0r, dega, degb)
    p0a, p0b = scat(gath(s0, row), col, zeros_d)
    s1 = _mm_mid(p0a, p0b, s0, W1, b1r, dega, degb)
    p1a, p1b = scat(gath(s1, row), col, zeros_d)
    s2 = _mm_mid(p1a, p1b, s1, W2, b2r, dega, degb)
    p2a, p2b = scat(gath(s2, row), col, zeros_d)

    out = _final(p2a, p2b, s2, dega, degb, batch3d, Wh1, bh1r, Wh2p, bh2p)
    return out[:G, 0]
